# baseline (speedup 1.0000x reference)
import functools
import math

import jax
import jax.numpy as jnp
from jax import lax
from jax.experimental import pallas as pl
from jax.experimental.pallas import tpu as pltpu
from jax.experimental.pallas import tpu_sc as plsc

D = 64
SCALE = math.sqrt(D)
L = 16
NC, NS = 2, 16
NW = NC * NS

CHUNK = 1024
IDX_W = 128
IDX_ROWS = CHUNK // IDX_W


def _emb_body(n_chunks, b_per_w, x_hbm, table_hbm, out_hbm,
              idx_v, rows_v, sem):
    wid = lax.axis_index("s") * NC + lax.axis_index("c")
    base = wid * b_per_w

    def chunk_body(g, carry):
        cbase = base + g * CHUNK
        irow = pl.multiple_of(cbase // IDX_W, 8)
        pltpu.sync_copy(x_hbm.at[pl.ds(irow, IDX_ROWS)], idx_v)
        cps = [
            pltpu.async_copy(table_hbm.at[idx_v.at[j]],
                             rows_v.at[pl.ds(j * IDX_W, IDX_W)], sem)
            for j in range(IDX_ROWS)
        ]
        for cp in cps:
            cp.wait()

        def scale_row(r, c2):
            for c in range(D // L):
                sl = pl.ds(c * L, L)
                rows_v[r, sl] = rows_v[r, sl] * SCALE
            return c2

        lax.fori_loop(0, CHUNK, scale_row, 0, unroll=2)

        pltpu.sync_copy(rows_v, out_hbm.at[pl.ds(cbase, CHUNK)])
        return carry

    lax.fori_loop(0, n_chunks, chunk_body, 0)


def kernel(x, table):
    b, h = x.shape
    n = b * h
    assert n % (NW * CHUNK) == 0
    b_per_w = n // NW
    n_chunks = b_per_w // CHUNK

    x_flat = x.reshape(n // IDX_W, IDX_W).astype(jnp.int32)

    mesh = plsc.VectorSubcoreMesh(core_axis_name="c", subcore_axis_name="s")
    emb = pl.kernel(
        functools.partial(_emb_body, n_chunks, b_per_w),
        mesh=mesh,
        compiler_params=pltpu.CompilerParams(use_tc_tiling_on_sc=False),
        out_type=jax.ShapeDtypeStruct((n, D), jnp.float32),
        scratch_types=[
            pltpu.VMEM((IDX_ROWS, IDX_W), jnp.int32),
            pltpu.VMEM((CHUNK, D), jnp.float32),
            pltpu.SemaphoreType.DMA,
        ],
    )
    out = emb(x_flat, table)
    return out.reshape(b, h, D)
